# baseline (speedup 1.0000x reference)
import jax
import jax.numpy as jnp
from jax.experimental import pallas as pl
from jax.experimental.pallas import tpu as pltpu

_SIZE = 65536
_DIM = 128
_BATCH = 1024
_WIDTHS = (2048, 2048, 4096, 8192, 16384, 32768)
_STARTS = tuple(sum(_WIDTHS[:j]) for j in range(len(_WIDTHS)))
_NCH = len(_WIDTHS)


def _body(out_hbm, bank_hbm, oo_hbm, snap_hbm, new_hbm, buf, vin, vout,
          isem, ssem, nsem, osem):
    def in_cp(j):
        c, w = _STARTS[j], _WIDTHS[j]
        return pltpu.make_async_copy(
            bank_hbm.at[:, pl.ds(c, w)], buf.at[:, pl.ds(c, w)], isem.at[j])

    def snap_cp(j):
        c, w = _STARTS[j], _WIDTHS[j]
        return pltpu.make_async_copy(
            buf.at[:, pl.ds(c, w)], snap_hbm.at[:, pl.ds(c, w)], ssem.at[j])

    def new_cp(j):
        c, w = _STARTS[j], _WIDTHS[j]
        if j == 0:
            c, w = _BATCH, _WIDTHS[0] - _BATCH
        return pltpu.make_async_copy(
            buf.at[:, pl.ds(c, w)], new_hbm.at[:, pl.ds(c, w)], nsem.at[j])

    ocp_in = pltpu.make_async_copy(out_hbm, vin, osem.at[0])
    ocp_in.start()
    in_cp(0).start()
    in_cp(1).start()
    ocp_in.wait()
    vout[...] = jnp.transpose(vin[...])
    pltpu.make_async_copy(vin, oo_hbm, osem.at[1]).start()
    pltpu.make_async_copy(vout, new_hbm.at[:, pl.ds(0, _BATCH)], osem.at[2]).start()
    for j in range(_NCH):
        in_cp(j).wait()
        if j + 2 < _NCH:
            in_cp(j + 2).start()
        snap_cp(j).start()
        new_cp(j).start()
    for j in range(_NCH):
        snap_cp(j).wait()
        new_cp(j).wait()
    pltpu.make_async_copy(vin, oo_hbm, osem.at[1]).wait()
    pltpu.make_async_copy(vout, new_hbm.at[:, pl.ds(0, _BATCH)], osem.at[2]).wait()


def kernel(output, bank):
    out_shapes = (
        jax.ShapeDtypeStruct((_BATCH, _DIM), output.dtype),
        jax.ShapeDtypeStruct((_DIM, _SIZE), bank.dtype),
        jax.ShapeDtypeStruct((_DIM, _SIZE), bank.dtype),
    )
    out, snap, new = pl.pallas_call(
        _body,
        in_specs=[
            pl.BlockSpec(memory_space=pl.ANY),
            pl.BlockSpec(memory_space=pl.ANY),
        ],
        out_specs=[
            pl.BlockSpec(memory_space=pl.ANY),
            pl.BlockSpec(memory_space=pl.ANY),
            pl.BlockSpec(memory_space=pl.ANY),
        ],
        out_shape=out_shapes,
        scratch_shapes=[
            pltpu.VMEM((_DIM, _SIZE), jnp.float32),
            pltpu.VMEM((_BATCH, _DIM), jnp.float32),
            pltpu.VMEM((_DIM, _BATCH), jnp.float32),
            pltpu.SemaphoreType.DMA((_NCH,)),
            pltpu.SemaphoreType.DMA((_NCH,)),
            pltpu.SemaphoreType.DMA((_NCH,)),
            pltpu.SemaphoreType.DMA((3,)),
        ],
    )(output, bank)
    return (out, snap, new)


# speedup vs baseline: 1.0195x; 1.0195x over previous
import jax
import jax.numpy as jnp
from jax.experimental import pallas as pl
from jax.experimental.pallas import tpu as pltpu

_SIZE = 65536
_DIM = 128
_BATCH = 1024
_WIDTHS = (16384, 49152)
_STARTS = tuple(sum(_WIDTHS[:j]) for j in range(len(_WIDTHS)))
_NCH = len(_WIDTHS)


def _body(out_hbm, bank_hbm, oo_hbm, snap_hbm, new_hbm, buf, vin, vout,
          isem, ssem, nsem, osem):
    def in_cp(j):
        c, w = _STARTS[j], _WIDTHS[j]
        return pltpu.make_async_copy(
            bank_hbm.at[:, pl.ds(c, w)], buf.at[:, pl.ds(c, w)], isem.at[j])

    def snap_cp(j):
        c, w = _STARTS[j], _WIDTHS[j]
        return pltpu.make_async_copy(
            buf.at[:, pl.ds(c, w)], snap_hbm.at[:, pl.ds(c, w)], ssem.at[j])

    def new_cp(j):
        c, w = _STARTS[j], _WIDTHS[j]
        if j == 0:
            c, w = _BATCH, _WIDTHS[0] - _BATCH
        return pltpu.make_async_copy(
            buf.at[:, pl.ds(c, w)], new_hbm.at[:, pl.ds(c, w)], nsem.at[j])

    ocp_in = pltpu.make_async_copy(out_hbm, vin, osem.at[0])
    ocp_in.start()
    in_cp(0).start()
    in_cp(1).start()
    ocp_in.wait()
    vout[...] = jnp.transpose(vin[...])
    pltpu.make_async_copy(vin, oo_hbm, osem.at[1]).start()
    pltpu.make_async_copy(vout, new_hbm.at[:, pl.ds(0, _BATCH)], osem.at[2]).start()
    for j in range(_NCH):
        in_cp(j).wait()
        if j + 2 < _NCH:
            in_cp(j + 2).start()
        snap_cp(j).start()
        new_cp(j).start()
    for j in range(_NCH):
        snap_cp(j).wait()
        new_cp(j).wait()
    pltpu.make_async_copy(vin, oo_hbm, osem.at[1]).wait()
    pltpu.make_async_copy(vout, new_hbm.at[:, pl.ds(0, _BATCH)], osem.at[2]).wait()


def kernel(output, bank):
    out_shapes = (
        jax.ShapeDtypeStruct((_BATCH, _DIM), output.dtype),
        jax.ShapeDtypeStruct((_DIM, _SIZE), bank.dtype),
        jax.ShapeDtypeStruct((_DIM, _SIZE), bank.dtype),
    )
    out, snap, new = pl.pallas_call(
        _body,
        in_specs=[
            pl.BlockSpec(memory_space=pl.ANY),
            pl.BlockSpec(memory_space=pl.ANY),
        ],
        out_specs=[
            pl.BlockSpec(memory_space=pl.ANY),
            pl.BlockSpec(memory_space=pl.ANY),
            pl.BlockSpec(memory_space=pl.ANY),
        ],
        out_shape=out_shapes,
        scratch_shapes=[
            pltpu.VMEM((_DIM, _SIZE), jnp.float32),
            pltpu.VMEM((_BATCH, _DIM), jnp.float32),
            pltpu.VMEM((_DIM, _BATCH), jnp.float32),
            pltpu.SemaphoreType.DMA((_NCH,)),
            pltpu.SemaphoreType.DMA((_NCH,)),
            pltpu.SemaphoreType.DMA((_NCH,)),
            pltpu.SemaphoreType.DMA((3,)),
        ],
    )(output, bank)
    return (out, snap, new)
